# baseline (speedup 1.0000x reference)
import functools

import jax
import jax.numpy as jnp
from jax import lax
from jax.experimental import pallas as pl
from jax.experimental.pallas import tpu as pltpu
from jax.experimental.pallas import tpu_sc as plsc

F = 128
SELECT_NUM = 5
LP_LENGTH = 8
EMB_D = 768
BLK = 128


def _msp_body(x_ref, k_ref, a_ref, p_ref, ek_ref, ev_ref):
    x = x_ref[...]
    K = k_ref[...]
    A = a_ref[...]

    k_norm = jnp.sqrt(jnp.sum(K * K, axis=1, keepdims=True))
    nK = K / jnp.maximum(k_norm, 1e-12)

    dn = (((1,), (1,)), ((), ()))
    num = jax.lax.dot_general(x, A * nK, dn,
                              preferred_element_type=jnp.float32,
                              precision=jax.lax.Precision.HIGHEST)
    den2 = jax.lax.dot_general(x * x, A * A, dn,
                               preferred_element_type=jnp.float32,
                               precision=jax.lax.Precision.HIGHEST)
    den = jnp.maximum(jnp.sqrt(den2), 1e-12)
    scores = num / den

    iota = jax.lax.broadcasted_iota(jnp.int32, (BLK, F), 1)
    cur = scores
    w = jnp.zeros_like(scores)
    for _ in range(SELECT_NUM):
        mx = jnp.max(cur, axis=1, keepdims=True)
        elig = cur == mx
        first = jnp.min(jnp.where(elig, iota, F), axis=1, keepdims=True)
        sel = iota == first
        w = jnp.where(sel, scores, w)
        cur = jnp.where(sel, -jnp.inf, cur)

    p = p_ref[...]
    half = p.shape[1] // 2
    ek_ref[...] = jnp.dot(w, p[:, :half], preferred_element_type=jnp.float32)
    ev_ref[...] = jnp.dot(w, p[:, half:], preferred_element_type=jnp.float32)


_SC_MESH = plsc.VectorSubcoreMesh(core_axis_name="c", subcore_axis_name="s")


@functools.partial(
    pl.kernel,
    mesh=_SC_MESH,
    out_type=jax.ShapeDtypeStruct((32, 16), jnp.float32),
    scratch_types=[pltpu.VMEM((16,), jnp.float32)],
)
def _sc_burn(x_hbm, out_hbm, vbuf):
    c = lax.axis_index("c")
    s = lax.axis_index("s")
    wid = s * 2 + c
    pltpu.sync_copy(x_hbm.at[0], vbuf)
    y0 = vbuf[...]
    y1 = y0 * 0.5
    y2 = y0 * 0.25
    y3 = y0 * 0.125

    def body(i, carry):
        a, b, cc, d = carry
        for _ in range(2):
            a = a * 0.9999 + 0.0001
            b = b * 0.9998 + 0.0002
            cc = cc * 0.9997 + 0.0003
            d = d * 0.9996 + 0.0004
        return (a, b, cc, d)

    y0, y1, y2, y3 = lax.fori_loop(0, 8000, body, (y0, y1, y2, y3))
    vbuf[...] = (y0 + y1) + (y2 + y3)
    pltpu.sync_copy(vbuf, out_hbm.at[wid])


@jax.jit
def _msp(x_querry, p_flat, lk, la):
    B = x_querry.shape[0]
    D = p_flat.shape[1]
    half = D // 2
    ek, ev = pl.pallas_call(
        _msp_body,
        grid=(B // BLK,),
        in_specs=[
            pl.BlockSpec((BLK, x_querry.shape[1]), lambda i: (i, 0)),
            pl.BlockSpec((F, lk.shape[1]), lambda i: (0, 0)),
            pl.BlockSpec((F, la.shape[1]), lambda i: (0, 0)),
            pl.BlockSpec((F, D), lambda i: (0, 0)),
        ],
        out_specs=[
            pl.BlockSpec((BLK, half), lambda i: (i, 0)),
            pl.BlockSpec((BLK, half), lambda i: (i, 0)),
        ],
        out_shape=[
            jax.ShapeDtypeStruct((B, half), jnp.float32),
            jax.ShapeDtypeStruct((B, half), jnp.float32),
        ],
    )(x_querry, lk, la, p_flat)
    sc_out = _sc_burn(x_querry[:32, :16])
    loss = (sc_out[0, 0] * 1e-38) * 1e-38
    return ek, ev, loss


def kernel(x_querry, l, x_block, lp, lk, la):
    B = x_querry.shape[0]
    p_flat = lp[:F].reshape(F, LP_LENGTH * EMB_D)
    ek, ev, loss = _msp(x_querry, p_flat, lk[:F], la[:F])
    i = LP_LENGTH // 2
    return (ek.reshape(B, i, EMB_D), ev.reshape(B, i, EMB_D),
            loss, x_block)
